# Initial kernel scaffold; baseline (speedup 1.0000x reference)
import jax
import jax.numpy as jnp
from jax.experimental import pallas as pl


def kernel(nuc_emebedding, f_node_label, f_node_assignment, f_message, node_graph, message_graph, scope, W_z, b_z, W_r, U_r_w, U_r_b, W_h, b_h, out_w, out_b):
    raise NotImplementedError("write your pallas kernel here")


# speedup vs baseline: 1.7383x; 1.7383x over previous
import functools

import jax
import jax.numpy as jnp
from jax import lax
from jax.experimental import pallas as pl
from jax.experimental.pallas import tpu as pltpu
from jax.experimental.pallas import tpu_sc as plsc

HID = 256
LBL = 4
DEPTH = 3
N_NUC = 60000
N_NODE = 20000
M_MSG = 40000
KN = 4
AS = 12
LANE = 16
NC = 2
NS = 16
NW = NC * NS

NODES_PER_W = 640
MSGS_PER_W = 1280
K1_C = 8
K3_C = 64
K5_C = 16


def _worker_id():
    return lax.axis_index("s") * NC + lax.axis_index("c")


def _fna_body(table, idx_flat, out, idx_v, rows_v, acc_v, sem):
    w = _worker_id()
    start = w * NODES_PER_W
    count = jnp.minimum(NODES_PER_W, jnp.maximum(N_NODE - start, 0))
    nchunks = count // K1_C

    def chunk(c, carry):
        nb = start + c * K1_C
        ib = pl.multiple_of(nb * AS, 8)
        pltpu.sync_copy(idx_flat.at[pl.ds(ib, K1_C * AS)], idx_v)
        pltpu.async_copy(table.at[idx_v], rows_v, sem).wait()

        def node(i, carry2):
            for cc in range(HID // LANE):
                sl = pl.ds(cc * LANE, LANE)
                s = rows_v[i * AS, sl]
                for j in range(1, AS):
                    s = s + rows_v[i * AS + j, sl]
                acc_v[i, sl] = s
            return carry2

        lax.fori_loop(0, K1_C, node, 0)
        pltpu.sync_copy(acc_v, out.at[pl.ds(nb, K1_C)])
        return carry

    lax.fori_loop(0, nchunks, chunk, 0)


def _lf_gather_body(tzh_n, r1_n, fmsg, azh_out, r1_out, idx_v, t_v, r_v, s1, s2):
    w = _worker_id()
    start = w * MSGS_PER_W
    count = jnp.minimum(MSGS_PER_W, jnp.maximum(M_MSG - start, 0))
    nchunks = count // K3_C

    def chunk(c, carry):
        mb = pl.multiple_of(start + c * K3_C, 8)
        pltpu.sync_copy(fmsg.at[pl.ds(mb, K3_C)], idx_v)
        cp1 = pltpu.async_copy(tzh_n.at[idx_v], t_v, s1)
        cp2 = pltpu.async_copy(r1_n.at[idx_v], r_v, s2)
        cp1.wait()
        cp2.wait()
        pltpu.sync_copy(t_v, azh_out.at[pl.ds(mb, K3_C)])
        pltpu.sync_copy(r_v, r1_out.at[pl.ds(mb, K3_C)])
        return carry

    lax.fori_loop(0, nchunks, chunk, 0)


def _edge_body(msg_t, p_t, r1_m, mg_flat, summ_out, sumg_out,
               idx_v, m_v, p_v, r1_v, s_v, g_v, s1, s2):
    w = _worker_id()
    start = w * MSGS_PER_W
    count = jnp.minimum(MSGS_PER_W, jnp.maximum(M_MSG - start, 0))
    nchunks = count // K5_C

    def chunk(c, carry):
        mb = pl.multiple_of(start + c * K5_C, 8)
        eb = pl.multiple_of(mb * KN, 8)
        pltpu.sync_copy(mg_flat.at[pl.ds(eb, K5_C * KN)], idx_v)
        cp1 = pltpu.async_copy(msg_t.at[idx_v], m_v, s1)
        cp2 = pltpu.async_copy(p_t.at[idx_v], p_v, s2)
        pltpu.sync_copy(r1_m.at[pl.ds(mb, K5_C)], r1_v)
        cp1.wait()
        cp2.wait()

        def msg_i(i, carry2):
            for cc in range(HID // LANE):
                sl = pl.ds(cc * LANE, LANE)
                r1c = r1_v[i, sl]
                m = m_v[i * KN, sl]
                p = p_v[i * KN, sl]
                sig = 1.0 / (1.0 + jnp.exp(-(r1c + p)))
                s = m
                g = sig * m
                for k in range(1, KN):
                    m = m_v[i * KN + k, sl]
                    p = p_v[i * KN + k, sl]
                    sig = 1.0 / (1.0 + jnp.exp(-(r1c + p)))
                    s = s + m
                    g = g + sig * m
                s_v[i, sl] = s
                g_v[i, sl] = g
            return carry2

        lax.fori_loop(0, K5_C, msg_i, 0)
        pltpu.sync_copy(s_v, summ_out.at[pl.ds(mb, K5_C)])
        pltpu.sync_copy(g_v, sumg_out.at[pl.ds(mb, K5_C)])
        return carry

    lax.fori_loop(0, nchunks, chunk, 0)


def _final_gather_body(msg_t, fna_t, ngf, i16,
                       inc_out, f16_out,
                       ng_v, mi_v, i_v, f_v, inc_v, sem):
    w = _worker_id()

    @pl.when(w == 0)
    def _():
        pltpu.sync_copy(ngf, ng_v)
        pltpu.async_copy(msg_t.at[ng_v], mi_v, sem).wait()
        pltpu.sync_copy(i16, i_v)
        pltpu.async_copy(fna_t.at[i_v], f_v, sem).wait()
        for i in range(16):
            for cc in range(HID // LANE):
                sl = pl.ds(cc * LANE, LANE)
                inc_v[i, sl] = (mi_v[i * KN, sl] + mi_v[i * KN + 1, sl]
                                + mi_v[i * KN + 2, sl] + mi_v[i * KN + 3, sl])
        pltpu.sync_copy(inc_v, inc_out)
        pltpu.sync_copy(f_v, f16_out)


R_NODE = 400
R_MSG = 400


def _node_mm_body(fna_ref, lbl_ref, wf1, wl1, b1, wf2, wl2, b2, tzh_ref, r1_ref):
    f = fna_ref[...]
    l = lbl_ref[...]
    tzh_ref[...] = (jnp.dot(f, wf1[...], preferred_element_type=jnp.float32)
                    + jnp.dot(l, wl1[...], preferred_element_type=jnp.float32)
                    + b1[...])
    r1_ref[...] = (jnp.dot(f, wf2[...], preferred_element_type=jnp.float32)
                   + jnp.dot(l, wl2[...], preferred_element_type=jnp.float32)
                   + b2[...])


def _row_mask(shape, pid):
    rows = lax.broadcasted_iota(jnp.int32, shape, 0) + pid * R_MSG
    return rows != 0


def _upd0_body(azh_ref, urt, msg_ref, p_ref):
    azh = azh_ref[...]
    m = jax.nn.sigmoid(azh[:, :HID]) * jnp.tanh(azh[:, HID:])
    m = jnp.where(_row_mask(m.shape, pl.program_id(0)), m, 0.0)
    msg_ref[...] = m
    p_ref[...] = jnp.dot(m, urt[...], preferred_element_type=jnp.float32)


def _upd_body(azh_ref, summ_ref, sumg_ref, wz2, wh2, urt, msg_ref, p_ref):
    azh = azh_ref[...]
    sm = summ_ref[...]
    sg = sumg_ref[...]
    z = jax.nn.sigmoid(azh[:, :HID]
                       + jnp.dot(sm, wz2[...], preferred_element_type=jnp.float32))
    h = jnp.tanh(azh[:, HID:]
                 + jnp.dot(sg, wh2[...], preferred_element_type=jnp.float32))
    m = (1.0 - z) * sm + z * h
    m = jnp.where(_row_mask(m.shape, pl.program_id(0)), m, 0.0)
    msg_ref[...] = m
    p_ref[...] = jnp.dot(m, urt[...], preferred_element_type=jnp.float32)


def _upd_final_body(azh_ref, summ_ref, sumg_ref, wz2, wh2, msg_ref):
    azh = azh_ref[...]
    sm = summ_ref[...]
    sg = sumg_ref[...]
    z = jax.nn.sigmoid(azh[:, :HID]
                       + jnp.dot(sm, wz2[...], preferred_element_type=jnp.float32))
    h = jnp.tanh(azh[:, HID:]
                 + jnp.dot(sg, wh2[...], preferred_element_type=jnp.float32))
    m = (1.0 - z) * sm + z * h
    msg_ref[...] = jnp.where(_row_mask(m.shape, pl.program_id(0)), m, 0.0)


def _out_body(f16, l16, inc16, wof, wol, woi, ob, out_ref):
    acc = (jnp.dot(f16[...], wof[...], preferred_element_type=jnp.float32)
           + jnp.dot(l16[...], wol[...], preferred_element_type=jnp.float32)
           + jnp.dot(inc16[...], woi[...], preferred_element_type=jnp.float32)
           + ob[...])
    out_ref[...] = jnp.maximum(acc, 0.0)


def _full(shape):
    return pl.BlockSpec(shape, lambda i: tuple(0 for _ in shape))


def _rows(shape):
    return pl.BlockSpec(shape, lambda i: (i,) + tuple(0 for _ in shape[1:]))


def kernel(nuc_emebedding, f_node_label, f_node_assignment, f_message,
           node_graph, message_graph, scope, W_z, b_z, W_r, U_r_w, U_r_b,
           W_h, b_h, out_w, out_b):
    f32 = jnp.float32
    i32 = jnp.int32

    table = jnp.concatenate(
        [nuc_emebedding, jnp.zeros((1, HID), f32)], axis=0)
    ia_flat = f_node_assignment.reshape(-1).astype(i32)
    fm = f_message.astype(i32)
    mg_flat = message_graph.reshape(-1).astype(i32)
    lbl_pad = jnp.pad(f_node_label, ((0, 0), (0, LANE - LBL)))
    idx16 = (scope[:, 0] + 1).astype(i32)
    ngf = node_graph[idx16].reshape(-1).astype(i32)

    HL = LBL + HID
    Wf1 = jnp.concatenate([W_z[:, LBL:HL].T, W_h[:, LBL:HL].T], axis=1)
    Wl1 = jnp.pad(jnp.concatenate([W_z[:, :LBL].T, W_h[:, :LBL].T], axis=1),
                  ((0, LANE - LBL), (0, 0)))
    b1 = jnp.concatenate([b_z, b_h])[None, :]
    Wf2 = W_r[:, LBL:].T
    Wl2 = jnp.pad(W_r[:, :LBL].T, ((0, LANE - LBL), (0, 0)))
    b2 = U_r_b[None, :]
    Wz2T = W_z[:, HL:].T
    Wh2T = W_h[:, HL:].T
    UrT = U_r_w.T
    Wol = jnp.pad(out_w[:, :LBL].T, ((0, LANE - LBL), (0, 0)))
    Wof = out_w[:, LBL:HL].T
    Woi = out_w[:, HL:].T
    ob = out_b[None, :]

    mesh = plsc.VectorSubcoreMesh(core_axis_name="c", subcore_axis_name="s",
                                  num_cores=NC, num_subcores=NS)

    fna = pl.kernel(
        _fna_body,
        out_type=jax.ShapeDtypeStruct((N_NODE, HID), f32),
        mesh=mesh,
        scratch_types=[
            pltpu.VMEM((K1_C * AS,), i32),
            pltpu.VMEM((K1_C * AS, HID), f32),
            pltpu.VMEM((K1_C, HID), f32),
            pltpu.SemaphoreType.DMA,
        ],
    )(table, ia_flat)

    tzh_n, r1_n = pl.pallas_call(
        _node_mm_body,
        grid=(N_NODE // R_NODE,),
        in_specs=[
            _rows((R_NODE, HID)), _rows((R_NODE, LANE)),
            _full((HID, 2 * HID)), _full((LANE, 2 * HID)), _full((1, 2 * HID)),
            _full((HID, HID)), _full((LANE, HID)), _full((1, HID)),
        ],
        out_specs=[_rows((R_NODE, 2 * HID)), _rows((R_NODE, HID))],
        out_shape=[jax.ShapeDtypeStruct((N_NODE, 2 * HID), f32),
                   jax.ShapeDtypeStruct((N_NODE, HID), f32)],
    )(fna, lbl_pad, Wf1, Wl1, b1, Wf2, Wl2, b2)

    azh, r1m = pl.kernel(
        _lf_gather_body,
        out_type=(jax.ShapeDtypeStruct((M_MSG, 2 * HID), f32),
                  jax.ShapeDtypeStruct((M_MSG, HID), f32)),
        mesh=mesh,
        scratch_types=[
            pltpu.VMEM((K3_C,), i32),
            pltpu.VMEM((K3_C, 2 * HID), f32),
            pltpu.VMEM((K3_C, HID), f32),
            pltpu.SemaphoreType.DMA,
            pltpu.SemaphoreType.DMA,
        ],
    )(tzh_n, r1_n, fm)

    msg, p = pl.pallas_call(
        _upd0_body,
        grid=(M_MSG // R_MSG,),
        in_specs=[_rows((R_MSG, 2 * HID)), _full((HID, HID))],
        out_specs=[_rows((R_MSG, HID)), _rows((R_MSG, HID))],
        out_shape=[jax.ShapeDtypeStruct((M_MSG, HID), f32),
                   jax.ShapeDtypeStruct((M_MSG, HID), f32)],
    )(azh, UrT)

    edge_call = pl.kernel(
        _edge_body,
        out_type=(jax.ShapeDtypeStruct((M_MSG, HID), f32),
                  jax.ShapeDtypeStruct((M_MSG, HID), f32)),
        mesh=mesh,
        scratch_types=[
            pltpu.VMEM((K5_C * KN,), i32),
            pltpu.VMEM((K5_C * KN, HID), f32),
            pltpu.VMEM((K5_C * KN, HID), f32),
            pltpu.VMEM((K5_C, HID), f32),
            pltpu.VMEM((K5_C, HID), f32),
            pltpu.VMEM((K5_C, HID), f32),
            pltpu.SemaphoreType.DMA,
            pltpu.SemaphoreType.DMA,
        ],
    )

    for t in range(1, DEPTH):
        summ, sumg = edge_call(msg, p, r1m, mg_flat)
        if t < DEPTH - 1:
            msg, p = pl.pallas_call(
                _upd_body,
                grid=(M_MSG // R_MSG,),
                in_specs=[
                    _rows((R_MSG, 2 * HID)), _rows((R_MSG, HID)),
                    _rows((R_MSG, HID)), _full((HID, HID)), _full((HID, HID)),
                    _full((HID, HID)),
                ],
                out_specs=[_rows((R_MSG, HID)), _rows((R_MSG, HID))],
                out_shape=[jax.ShapeDtypeStruct((M_MSG, HID), f32),
                           jax.ShapeDtypeStruct((M_MSG, HID), f32)],
            )(azh, summ, sumg, Wz2T, Wh2T, UrT)
        else:
            msg = pl.pallas_call(
                _upd_final_body,
                grid=(M_MSG // R_MSG,),
                in_specs=[
                    _rows((R_MSG, 2 * HID)), _rows((R_MSG, HID)),
                    _rows((R_MSG, HID)), _full((HID, HID)), _full((HID, HID)),
                ],
                out_specs=_rows((R_MSG, HID)),
                out_shape=jax.ShapeDtypeStruct((M_MSG, HID), f32),
            )(azh, summ, sumg, Wz2T, Wh2T)

    inc16, f16 = pl.kernel(
        _final_gather_body,
        out_type=(jax.ShapeDtypeStruct((16, HID), f32),
                  jax.ShapeDtypeStruct((16, HID), f32)),
        mesh=mesh,
        scratch_types=[
            pltpu.VMEM((16 * KN,), i32),
            pltpu.VMEM((16 * KN, HID), f32),
            pltpu.VMEM((16,), i32),
            pltpu.VMEM((16, HID), f32),
            pltpu.VMEM((16, HID), f32),
            pltpu.SemaphoreType.DMA,
        ],
    )(msg, fna, ngf, idx16)
    l16 = lbl_pad[idx16]

    batch_hpn_vec = pl.pallas_call(
        _out_body,
        grid=(1,),
        in_specs=[_full((16, HID)), _full((16, LANE)), _full((16, HID)),
                  _full((HID, HID)), _full((LANE, HID)), _full((HID, HID)),
                  _full((1, HID))],
        out_specs=_full((16, HID)),
        out_shape=jax.ShapeDtypeStruct((16, HID), f32),
    )(f16, l16, inc16, Wof, Wol, Woi, ob)

    return (msg, batch_hpn_vec)
